# Initial kernel scaffold; baseline (speedup 1.0000x reference)
import jax
import jax.numpy as jnp
from jax.experimental import pallas as pl


def kernel(x, mask, n_nodes):
    raise NotImplementedError("write your pallas kernel here")


# speedup vs baseline: 1.0434x; 1.0434x over previous
import functools

import jax
import jax.numpy as jnp
from jax import lax
from jax.experimental import pallas as pl
from jax.experimental.pallas import tpu as pltpu
from jax.experimental.pallas import tpu_sc as plsc

NC = 2
NS = 16
NW = NC * NS


def _gather_body(B, BPW, x_hbm, mask_hbm, idx_hbm, xout_hbm, mout_hbm,
                 idx_v, rows_v, mval_v, sem_x, sem_m):
    wid = lax.axis_index("s") * NC + lax.axis_index("c")
    base = wid * BPW
    last = B - (NW - 1) * BPW

    pltpu.sync_copy(idx_hbm.at[pl.ds(base, BPW)], idx_v)
    cp_x = pltpu.async_copy(x_hbm.at[idx_v], rows_v, sem_x)
    cp_m = pltpu.async_copy(mask_hbm.at[idx_v], mval_v, sem_m)
    cp_x.wait()
    cp_m.wait()

    @pl.when(wid != NW - 1)
    def _():
        pltpu.sync_copy(rows_v, xout_hbm.at[pl.ds(base, BPW)])
        pltpu.sync_copy(mval_v, mout_hbm.at[pl.ds(base, BPW)])

    @pl.when(wid == NW - 1)
    def _():
        pltpu.sync_copy(rows_v.at[pl.ds(0, last)],
                        xout_hbm.at[pl.ds(base, last)])
        pltpu.sync_copy(mval_v.at[pl.ds(0, last)],
                        mout_hbm.at[pl.ds(base, last)])


@functools.partial(jax.jit, static_argnames=("B", "D", "BPW"))
def _sc_gather(x, mask_i32, idx_pad, B, D, BPW):
    mesh = plsc.VectorSubcoreMesh(core_axis_name="c", subcore_axis_name="s")
    fn = pl.kernel(
        functools.partial(_gather_body, B, BPW),
        out_type=[
            jax.ShapeDtypeStruct((B, D), jnp.float32),
            jax.ShapeDtypeStruct((B,), jnp.int32),
        ],
        mesh=mesh,
        scratch_types=[
            pltpu.VMEM((BPW,), jnp.int32),
            pltpu.VMEM((BPW, D), jnp.float32),
            pltpu.VMEM((BPW,), jnp.int32),
            pltpu.SemaphoreType.DMA,
            pltpu.SemaphoreType.DMA,
        ],
    )
    return fn(x, mask_i32, idx_pad)


def kernel(x, mask, n_nodes):
    B = n_nodes.shape[0]
    D = x.shape[1]
    BPW = ((B + NW - 1) // NW + 7) // 8 * 8
    pad = NW * BPW - B
    idx_pad = jnp.concatenate(
        [n_nodes.astype(jnp.int32), jnp.zeros((pad,), jnp.int32)])
    mask_i32 = mask.astype(jnp.int32)
    x_nodes, mask_nodes_i32 = _sc_gather(x, mask_i32, idx_pad, B, D, BPW)
    return (x_nodes, x, mask_nodes_i32 != 0, mask)
